# baseline (speedup 1.0000x reference)
import jax
import jax.numpy as jnp
import numpy as np
from jax import lax
from jax.experimental import pallas as pl
from jax.experimental.pallas import tpu as pltpu
from jax.experimental.pallas import tpu_sc as plsc

_SKELETON = np.array(
    [[15, 13], [13, 11], [16, 14], [14, 12], [11, 12], [5, 11], [6, 12],
     [5, 6], [5, 7], [6, 8], [7, 9], [8, 10], [1, 2], [0, 1], [0, 2],
     [1, 3], [2, 4], [3, 5], [4, 6]], dtype=np.int32)
_P = 30
_PP = 32
_S = 10
_PEAK_THRESH = 0.1
_PAF_SCORE_THRESH = 0.05
_PAF_COUNT_THRESH = 0.8
_NEG = -1e9


def _peaks_kernel(h_ref, tv_ref, ti_ref, nb_ref):
    h = h_ref[0]
    K, H, W = h.shape
    rows = lax.broadcasted_iota(jnp.int32, (K, H, W), 1)
    cols = lax.broadcasted_iota(jnp.int32, (K, H, W), 2)
    ninf = jnp.float32(-jnp.inf)

    up = jnp.where(rows < H - 1, jnp.roll(h, -1, axis=1), ninf)
    dn = jnp.where(rows > 0, jnp.roll(h, 1, axis=1), ninf)
    vmax = jnp.maximum(jnp.maximum(h, up), dn)
    rg = jnp.where(cols < W - 1, jnp.roll(vmax, -1, axis=2), ninf)
    lf = jnp.where(cols > 0, jnp.roll(vmax, 1, axis=2), ninf)
    maxf = jnp.maximum(jnp.maximum(vmax, rg), lf)
    is_peak = (h == maxf) & (h > _PEAK_THRESH)
    masked0 = jnp.where(is_peak, h, jnp.float32(_NEG))

    lin = rows * W + cols
    colio = lax.broadcasted_iota(jnp.int32, (K, _PP), 1)
    zf = jnp.zeros((K, _PP), jnp.float32)
    zi = jnp.zeros((K, _PP), jnp.int32)

    def body(i, carry):
        masked, tv, ti = carry
        m = jnp.max(masked, axis=(1, 2), keepdims=True)
        cand = jnp.where(masked == m, lin, jnp.int32(1 << 30))
        idx = jnp.min(cand, axis=(1, 2), keepdims=True)
        tv = jnp.where(colio == i, m[:, :, 0], tv)
        ti = jnp.where(colio == i, idx[:, :, 0], ti)
        masked = jnp.where(cand == idx, jnp.float32(-2e9), masked)
        return masked, tv, ti

    _, tv, ti = lax.fori_loop(0, _P, body, (masked0, zf, zi))

    ys = ti // W
    xs = ti % W
    nb_ref[0, :, 0] = ti
    nb_ref[0, :, 1] = ys * W + jnp.minimum(xs + 1, W - 1)
    nb_ref[0, :, 2] = ys * W + jnp.maximum(xs - 1, 0)
    nb_ref[0, :, 3] = jnp.minimum(ys + 1, H - 1) * W + xs
    nb_ref[0, :, 4] = jnp.maximum(ys - 1, 0) * W + xs
    tv_ref[0] = tv
    ti_ref[0] = ti


def _subpix_kernel(ti_ref, g_ref, px_ref, py_ref):
    W = 128
    H = 128
    ti = ti_ref[...]
    ys = ti // W
    xs = ti % W
    c0 = g_ref[:, :, 0]
    r = g_ref[:, :, 1]
    l = g_ref[:, :, 2]
    d = g_ref[:, :, 3]
    u = g_ref[:, :, 4]
    dx0 = 0.5 * (r - l)
    dy0 = 0.5 * (d - u)
    dxx = r + l - 2.0 * c0
    dyy = d + u - 2.0 * c0
    offx = jnp.where(dxx != 0, dx0 / (-jnp.where(dxx != 0, dxx, 1.0)), dx0)
    offy = jnp.where(dyy != 0, dy0 / (-jnp.where(dyy != 0, dyy, 1.0)), dy0)
    interior = (xs > 0) & (xs < W - 1) & (ys > 0) & (ys < H - 1)
    px_ref[...] = xs.astype(jnp.float32) + jnp.where(interior, offx, 0.0)
    py_ref[...] = ys.astype(jnp.float32) + jnp.where(interior, offy, 0.0)


def _paf_idx_kernel(pk_ref, t_ref, idx_ref):
    pk = pk_ref[0]
    axg, ayg = pk[:, 0], pk[:, 1]
    bxg, byg = pk[:, 3], pk[:, 4]
    H = W = 128

    dxp = bxg - axg
    dyp = byg - ayg
    for s in range(_S):
        ts = t_ref[0, s]
        sxf = axg + ts * dxp
        syf = ayg + ts * dyp
        sx = jnp.clip(jnp.floor(sxf).astype(jnp.int32), 0, W - 1)
        sy = jnp.clip(jnp.floor(syf).astype(jnp.int32), 0, H - 1)
        idx_ref[0, :, s] = sy * W + sx


def _sc_gather(tab, idx, chunks):
    NT, TW = tab.shape
    NI = idx.shape[1]
    CW = NI // chunks
    mesh = plsc.VectorSubcoreMesh(core_axis_name="core",
                                  subcore_axis_name="subcore")

    @pl.kernel(out_type=jax.ShapeDtypeStruct((NT, NI), tab.dtype), mesh=mesh,
               compiler_params=pltpu.CompilerParams(needs_layout_passes=False))
    def gather_kernel(tab_hbm, i_hbm, o_hbm):
        def body(t_vmem, i_vmem, o_vmem):
            tv = t_vmem.at[0]

            @pl.loop(0, CW // 16)
            def _(k):
                iv = i_vmem[0, pl.ds(k * 16, 16)]
                o_vmem[0, pl.ds(k * 16, 16)] = plsc.load_gather(tv, [iv])

        pltpu.emit_pipeline(
            body,
            grid=(NT, chunks),
            in_specs=[
                pl.BlockSpec((1, TW), index_map=lambda i, j: (i, 0)),
                pl.BlockSpec((1, CW), index_map=lambda i, j: (i, j)),
            ],
            out_specs=[pl.BlockSpec((1, CW), index_map=lambda i, j: (i, j))],
            core_axis_name=("core", "subcore"),
            dimension_semantics=(pltpu.PARALLEL, pltpu.PARALLEL),
        )(tab_hbm, i_hbm, o_hbm)

    return gather_kernel(tab, idx)


def _sc_gather2(tab, idx, chunks):
    NT, NI = idx.shape
    TW = tab.shape[1]
    CW = NI // chunks
    mesh = plsc.VectorSubcoreMesh(core_axis_name="core",
                                  subcore_axis_name="subcore")

    @pl.kernel(out_type=[jax.ShapeDtypeStruct((NT, NI), tab.dtype)] * 2,
               mesh=mesh,
               compiler_params=pltpu.CompilerParams(needs_layout_passes=False))
    def gather_kernel(tab_hbm, i_hbm, ox_hbm, oy_hbm):
        def body(tx_vmem, ty_vmem, i_vmem, ox_vmem, oy_vmem):
            tx = tx_vmem.at[0]
            ty = ty_vmem.at[0]

            @pl.loop(0, CW // 16)
            def _(k):
                iv = i_vmem[0, pl.ds(k * 16, 16)]
                ox_vmem[0, pl.ds(k * 16, 16)] = plsc.load_gather(tx, [iv])
                oy_vmem[0, pl.ds(k * 16, 16)] = plsc.load_gather(ty, [iv])

        pltpu.emit_pipeline(
            body,
            grid=(NT, chunks),
            in_specs=[
                pl.BlockSpec((1, TW), index_map=lambda i, j: (2 * i, 0)),
                pl.BlockSpec((1, TW), index_map=lambda i, j: (2 * i + 1, 0)),
                pl.BlockSpec((1, CW), index_map=lambda i, j: (i, j)),
            ],
            out_specs=[
                pl.BlockSpec((1, CW), index_map=lambda i, j: (i, j)),
                pl.BlockSpec((1, CW), index_map=lambda i, j: (i, j)),
            ],
            core_axis_name=("core", "subcore"),
            dimension_semantics=(pltpu.PARALLEL, pltpu.PARALLEL),
        )(tab_hbm, tab_hbm, i_hbm, ox_hbm, oy_hbm)

    return gather_kernel(tab, idx)


def _paf_score_kernel(pk_ref, gx_ref, gy_ref, out_ref):
    pk = pk_ref[0]
    axg, ayg, sag = pk[:, 0], pk[:, 1], pk[:, 2]
    bxg, byg, sbg = pk[:, 3], pk[:, 4], pk[:, 5]

    dxp = bxg - axg
    dyp = byg - ayg
    norm = jnp.sqrt(dxp * dxp + dyp * dyp) + 1e-8
    vx = dxp / norm
    vy = dyp / norm

    gx = gx_ref[0]
    gy = gy_ref[0]
    sum_vec = jnp.zeros_like(vx)
    cnt = jnp.zeros_like(vx)
    for s in range(_S):
        vec = gx[:, s] * vx + gy[:, s] * vy
        sum_vec = sum_vec + vec
        cnt = cnt + jnp.where(vec > _PAF_SCORE_THRESH, 1.0, 0.0)

    mean = sum_vec / jnp.float32(_S)
    ratio = cnt / jnp.float32(_S)
    va = sag > _PEAK_THRESH
    vb = sbg > _PEAK_THRESH
    pair_ok = (mean > 0) & (ratio > _PAF_COUNT_THRESH) & va & vb
    conn = jnp.where(pair_ok, mean + 0.5 * (sag + sbg), jnp.float32(_NEG))
    out_ref[0] = conn


def kernel(heat_pred, paf_pred):
    B = heat_pred.shape[0]
    h = B // 4
    outs = [_half(heat_pred[i * h:(i + 1) * h], paf_pred[i * h:(i + 1) * h])
            for i in range(4)]
    return tuple(jnp.concatenate(parts, axis=0) for parts in zip(*outs))


def _half(heat_pred, paf_pred):
    B, K, H, W = heat_pred.shape
    C = _SKELETON.shape[0]

    tv_p, ti_p, nb = pl.pallas_call(
        _peaks_kernel,
        grid=(B,),
        in_specs=[pl.BlockSpec((1, K, H, W), lambda b: (b, 0, 0, 0))],
        out_specs=[
            pl.BlockSpec((1, K, _PP), lambda b: (b, 0, 0)),
            pl.BlockSpec((1, K, _PP), lambda b: (b, 0, 0)),
            pl.BlockSpec((1, K, 5, _PP), lambda b: (b, 0, 0, 0)),
        ],
        out_shape=[
            jax.ShapeDtypeStruct((B, K, _PP), jnp.float32),
            jax.ShapeDtypeStruct((B, K, _PP), jnp.int32),
            jax.ShapeDtypeStruct((B, K, 5, _PP), jnp.int32),
        ],
        compiler_params=pltpu.CompilerParams(
            dimension_semantics=("parallel",)),
    )(heat_pred)

    heat_tab = heat_pred.reshape(B * K, H * W)
    nbg = _sc_gather(heat_tab, nb.reshape(B * K, 5 * _PP), 1)
    nbg = nbg.reshape(B, K, 5, _PP)

    px_p, py_p = pl.pallas_call(
        _subpix_kernel,
        out_shape=[jax.ShapeDtypeStruct((B, K, _PP), jnp.float32)] * 2,
    )(ti_p, nbg)

    px = px_p[:, :, :_P]
    py = py_p[:, :, :_P]
    topv = tv_p[:, :, :_P]
    valid = topv > _PEAK_THRESH

    p_lin = np.arange(8 * 128)
    ia = np.minimum(p_lin // _P, _P - 1).reshape(8, 128)
    ib = (p_lin % _P).reshape(8, 128)
    a_k = _SKELETON[:, 0]
    b_k = _SKELETON[:, 1]
    ax = px[:, a_k][:, :, ia]
    ay = py[:, a_k][:, :, ia]
    sa = topv[:, a_k][:, :, ia]
    bx = px[:, b_k][:, :, ib]
    by = py[:, b_k][:, :, ib]
    sb = topv[:, b_k][:, :, ib]
    pk = jnp.stack([ax, ay, sa, bx, by, sb], axis=2)

    t = jnp.linspace(0.0, 1.0, _S).reshape(1, _S)

    idx = pl.pallas_call(
        _paf_idx_kernel,
        grid=(B,),
        in_specs=[
            pl.BlockSpec((1, C, 6, 8, 128), lambda b: (b, 0, 0, 0, 0)),
            pl.BlockSpec((1, _S), lambda b: (0, 0)),
        ],
        out_specs=pl.BlockSpec((1, C, _S, 8, 128),
                               lambda b: (b, 0, 0, 0, 0)),
        out_shape=jax.ShapeDtypeStruct((B, C, _S, 8, 128), jnp.int32),
        compiler_params=pltpu.CompilerParams(
            dimension_semantics=("parallel",)),
    )(pk, t)

    table = paf_pred.reshape(B * C * 2, H * W)
    gxf, gyf = _sc_gather2(table, idx.reshape(B * C, _S * 8 * 128), 2)
    gx = gxf.reshape(B, C, _S, 8, 128)
    gy = gyf.reshape(B, C, _S, 8, 128)

    conn_p = pl.pallas_call(
        _paf_score_kernel,
        grid=(B,),
        in_specs=[
            pl.BlockSpec((1, C, 6, 8, 128), lambda b: (b, 0, 0, 0, 0)),
            pl.BlockSpec((1, C, _S, 8, 128), lambda b: (b, 0, 0, 0, 0)),
            pl.BlockSpec((1, C, _S, 8, 128), lambda b: (b, 0, 0, 0, 0)),
        ],
        out_specs=pl.BlockSpec((1, C, 8, 128), lambda b: (b, 0, 0, 0)),
        out_shape=jax.ShapeDtypeStruct((B, C, 8, 128), jnp.float32),
        compiler_params=pltpu.CompilerParams(
            dimension_semantics=("parallel",)),
    )(pk, gx, gy)

    conn = conn_p.reshape(B, C, 8 * 128)[:, :, :_P * _P].reshape(B, C, _P, _P)
    return px, py, topv, valid, conn


# speedup vs baseline: 1.0630x; 1.0630x over previous
import jax
import jax.numpy as jnp
import numpy as np
from jax import lax
from jax.experimental import pallas as pl
from jax.experimental.pallas import tpu as pltpu
from jax.experimental.pallas import tpu_sc as plsc

_SKELETON = np.array(
    [[15, 13], [13, 11], [16, 14], [14, 12], [11, 12], [5, 11], [6, 12],
     [5, 6], [5, 7], [6, 8], [7, 9], [8, 10], [1, 2], [0, 1], [0, 2],
     [1, 3], [2, 4], [3, 5], [4, 6]], dtype=np.int32)
_P = 30
_PP = 32
_S = 10
_PEAK_THRESH = 0.1
_PAF_SCORE_THRESH = 0.05
_PAF_COUNT_THRESH = 0.8
_NEG = -1e9


def _peaks_kernel(h_ref, tv_ref, ti_ref, nb_ref):
    h = h_ref[0]
    K, H, W = h.shape
    rows = lax.broadcasted_iota(jnp.int32, (K, H, W), 1)
    cols = lax.broadcasted_iota(jnp.int32, (K, H, W), 2)
    ninf = jnp.float32(-jnp.inf)

    up = jnp.where(rows < H - 1, jnp.roll(h, -1, axis=1), ninf)
    dn = jnp.where(rows > 0, jnp.roll(h, 1, axis=1), ninf)
    vmax = jnp.maximum(jnp.maximum(h, up), dn)
    rg = jnp.where(cols < W - 1, jnp.roll(vmax, -1, axis=2), ninf)
    lf = jnp.where(cols > 0, jnp.roll(vmax, 1, axis=2), ninf)
    maxf = jnp.maximum(jnp.maximum(vmax, rg), lf)
    is_peak = (h == maxf) & (h > _PEAK_THRESH)
    masked0 = jnp.where(is_peak, h, jnp.float32(_NEG))

    lin = rows * W + cols
    colio = lax.broadcasted_iota(jnp.int32, (K, _PP), 1)
    zf = jnp.zeros((K, _PP), jnp.float32)
    zi = jnp.zeros((K, _PP), jnp.int32)

    def body(i, carry):
        masked, tv, ti = carry
        m = jnp.max(masked, axis=(1, 2), keepdims=True)
        cand = jnp.where(masked == m, lin, jnp.int32(1 << 30))
        idx = jnp.min(cand, axis=(1, 2), keepdims=True)
        tv = jnp.where(colio == i, m[:, :, 0], tv)
        ti = jnp.where(colio == i, idx[:, :, 0], ti)
        masked = jnp.where(cand == idx, jnp.float32(-2e9), masked)
        return masked, tv, ti

    _, tv, ti = lax.fori_loop(0, _P, body, (masked0, zf, zi))

    ys = ti // W
    xs = ti % W
    nb_ref[0, :, 0] = ti
    nb_ref[0, :, 1] = ys * W + jnp.minimum(xs + 1, W - 1)
    nb_ref[0, :, 2] = ys * W + jnp.maximum(xs - 1, 0)
    nb_ref[0, :, 3] = jnp.minimum(ys + 1, H - 1) * W + xs
    nb_ref[0, :, 4] = jnp.maximum(ys - 1, 0) * W + xs
    tv_ref[0] = tv
    ti_ref[0] = ti


def _subpix_kernel(ti_ref, g_ref, px_ref, py_ref):
    W = 128
    H = 128
    ti = ti_ref[...]
    ys = ti // W
    xs = ti % W
    c0 = g_ref[:, :, 0]
    r = g_ref[:, :, 1]
    l = g_ref[:, :, 2]
    d = g_ref[:, :, 3]
    u = g_ref[:, :, 4]
    dx0 = 0.5 * (r - l)
    dy0 = 0.5 * (d - u)
    dxx = r + l - 2.0 * c0
    dyy = d + u - 2.0 * c0
    offx = jnp.where(dxx != 0, dx0 / (-jnp.where(dxx != 0, dxx, 1.0)), dx0)
    offy = jnp.where(dyy != 0, dy0 / (-jnp.where(dyy != 0, dyy, 1.0)), dy0)
    interior = (xs > 0) & (xs < W - 1) & (ys > 0) & (ys < H - 1)
    px_ref[...] = xs.astype(jnp.float32) + jnp.where(interior, offx, 0.0)
    py_ref[...] = ys.astype(jnp.float32) + jnp.where(interior, offy, 0.0)


def _paf_idx_kernel(pk_ref, t_ref, idx_ref):
    pk = pk_ref[0]
    axg, ayg = pk[:, 0], pk[:, 1]
    bxg, byg = pk[:, 3], pk[:, 4]
    H = W = 128

    dxp = bxg - axg
    dyp = byg - ayg
    for s in range(_S):
        ts = t_ref[0, s]
        sxf = axg + ts * dxp
        syf = ayg + ts * dyp
        sx = jnp.clip(jnp.floor(sxf).astype(jnp.int32), 0, W - 1)
        sy = jnp.clip(jnp.floor(syf).astype(jnp.int32), 0, H - 1)
        idx_ref[0, :, s] = sy * W + sx


def _sc_gather(tab, idx, chunks):
    NT, TW = tab.shape
    NI = idx.shape[1]
    CW = NI // chunks
    mesh = plsc.VectorSubcoreMesh(core_axis_name="core",
                                  subcore_axis_name="subcore")

    @pl.kernel(out_type=jax.ShapeDtypeStruct((NT, NI), tab.dtype), mesh=mesh,
               compiler_params=pltpu.CompilerParams(needs_layout_passes=False))
    def gather_kernel(tab_hbm, i_hbm, o_hbm):
        def body(t_vmem, i_vmem, o_vmem):
            tv = t_vmem.at[0]

            @pl.loop(0, CW // 16)
            def _(k):
                iv = i_vmem[0, pl.ds(k * 16, 16)]
                o_vmem[0, pl.ds(k * 16, 16)] = plsc.load_gather(tv, [iv])

        pltpu.emit_pipeline(
            body,
            grid=(NT, chunks),
            in_specs=[
                pl.BlockSpec((1, TW), index_map=lambda i, j: (i, 0)),
                pl.BlockSpec((1, CW), index_map=lambda i, j: (i, j)),
            ],
            out_specs=[pl.BlockSpec((1, CW), index_map=lambda i, j: (i, j))],
            core_axis_name=("core", "subcore"),
            dimension_semantics=(pltpu.PARALLEL, pltpu.PARALLEL),
        )(tab_hbm, i_hbm, o_hbm)

    return gather_kernel(tab, idx)


def _sc_gather2(tab, idx, chunks):
    NT, NI = idx.shape
    TW = tab.shape[1]
    CW = NI // chunks
    mesh = plsc.VectorSubcoreMesh(core_axis_name="core",
                                  subcore_axis_name="subcore")

    @pl.kernel(out_type=[jax.ShapeDtypeStruct((NT, NI), tab.dtype)] * 2,
               mesh=mesh,
               compiler_params=pltpu.CompilerParams(needs_layout_passes=False))
    def gather_kernel(tab_hbm, i_hbm, ox_hbm, oy_hbm):
        def body(tx_vmem, ty_vmem, i_vmem, ox_vmem, oy_vmem):
            tx = tx_vmem.at[0]
            ty = ty_vmem.at[0]

            @pl.loop(0, CW // 16)
            def _(k):
                iv = i_vmem[0, pl.ds(k * 16, 16)]
                ox_vmem[0, pl.ds(k * 16, 16)] = plsc.load_gather(tx, [iv])
                oy_vmem[0, pl.ds(k * 16, 16)] = plsc.load_gather(ty, [iv])

        pltpu.emit_pipeline(
            body,
            grid=(NT, chunks),
            in_specs=[
                pl.BlockSpec((1, TW), index_map=lambda i, j: (2 * i, 0)),
                pl.BlockSpec((1, TW), index_map=lambda i, j: (2 * i + 1, 0)),
                pl.BlockSpec((1, CW), index_map=lambda i, j: (i, j)),
            ],
            out_specs=[
                pl.BlockSpec((1, CW), index_map=lambda i, j: (i, j)),
                pl.BlockSpec((1, CW), index_map=lambda i, j: (i, j)),
            ],
            core_axis_name=("core", "subcore"),
            dimension_semantics=(pltpu.PARALLEL, pltpu.PARALLEL),
        )(tab_hbm, tab_hbm, i_hbm, ox_hbm, oy_hbm)

    return gather_kernel(tab, idx)


def _paf_score_kernel(pk_ref, gx_ref, gy_ref, out_ref):
    pk = pk_ref[0]
    axg, ayg, sag = pk[:, 0], pk[:, 1], pk[:, 2]
    bxg, byg, sbg = pk[:, 3], pk[:, 4], pk[:, 5]

    dxp = bxg - axg
    dyp = byg - ayg
    norm = jnp.sqrt(dxp * dxp + dyp * dyp) + 1e-8
    vx = dxp / norm
    vy = dyp / norm

    gx = gx_ref[0]
    gy = gy_ref[0]
    sum_vec = jnp.zeros_like(vx)
    cnt = jnp.zeros_like(vx)
    for s in range(_S):
        vec = gx[:, s] * vx + gy[:, s] * vy
        sum_vec = sum_vec + vec
        cnt = cnt + jnp.where(vec > _PAF_SCORE_THRESH, 1.0, 0.0)

    mean = sum_vec / jnp.float32(_S)
    ratio = cnt / jnp.float32(_S)
    va = sag > _PEAK_THRESH
    vb = sbg > _PEAK_THRESH
    pair_ok = (mean > 0) & (ratio > _PAF_COUNT_THRESH) & va & vb
    conn = jnp.where(pair_ok, mean + 0.5 * (sag + sbg), jnp.float32(_NEG))
    out_ref[0] = conn


def kernel(heat_pred, paf_pred):
    B = heat_pred.shape[0]
    h = B // 2
    o1 = _half(heat_pred[:h], paf_pred[:h])
    o2 = _half(heat_pred[h:], paf_pred[h:])
    return tuple(jnp.concatenate([a, b], axis=0) for a, b in zip(o1, o2))


def _half(heat_pred, paf_pred):
    B, K, H, W = heat_pred.shape
    C = _SKELETON.shape[0]

    tv_p, ti_p, nb = pl.pallas_call(
        _peaks_kernel,
        grid=(B,),
        in_specs=[pl.BlockSpec((1, K, H, W), lambda b: (b, 0, 0, 0))],
        out_specs=[
            pl.BlockSpec((1, K, _PP), lambda b: (b, 0, 0)),
            pl.BlockSpec((1, K, _PP), lambda b: (b, 0, 0)),
            pl.BlockSpec((1, K, 5, _PP), lambda b: (b, 0, 0, 0)),
        ],
        out_shape=[
            jax.ShapeDtypeStruct((B, K, _PP), jnp.float32),
            jax.ShapeDtypeStruct((B, K, _PP), jnp.int32),
            jax.ShapeDtypeStruct((B, K, 5, _PP), jnp.int32),
        ],
        compiler_params=pltpu.CompilerParams(
            dimension_semantics=("parallel",)),
    )(heat_pred)

    heat_tab = heat_pred.reshape(B * K, H * W)
    nbg = _sc_gather(heat_tab, nb.reshape(B * K, 5 * _PP), 1)
    nbg = nbg.reshape(B, K, 5, _PP)

    px_p, py_p = pl.pallas_call(
        _subpix_kernel,
        out_shape=[jax.ShapeDtypeStruct((B, K, _PP), jnp.float32)] * 2,
    )(ti_p, nbg)

    px = px_p[:, :, :_P]
    py = py_p[:, :, :_P]
    topv = tv_p[:, :, :_P]
    valid = topv > _PEAK_THRESH

    p_lin = np.arange(8 * 128)
    ia = np.minimum(p_lin // _P, _P - 1).reshape(8, 128)
    ib = (p_lin % _P).reshape(8, 128)
    a_k = _SKELETON[:, 0]
    b_k = _SKELETON[:, 1]
    ax = px[:, a_k][:, :, ia]
    ay = py[:, a_k][:, :, ia]
    sa = topv[:, a_k][:, :, ia]
    bx = px[:, b_k][:, :, ib]
    by = py[:, b_k][:, :, ib]
    sb = topv[:, b_k][:, :, ib]
    pk = jnp.stack([ax, ay, sa, bx, by, sb], axis=2)

    t = jnp.linspace(0.0, 1.0, _S).reshape(1, _S)

    idx = pl.pallas_call(
        _paf_idx_kernel,
        grid=(B,),
        in_specs=[
            pl.BlockSpec((1, C, 6, 8, 128), lambda b: (b, 0, 0, 0, 0)),
            pl.BlockSpec((1, _S), lambda b: (0, 0)),
        ],
        out_specs=pl.BlockSpec((1, C, _S, 8, 128),
                               lambda b: (b, 0, 0, 0, 0)),
        out_shape=jax.ShapeDtypeStruct((B, C, _S, 8, 128), jnp.int32),
        compiler_params=pltpu.CompilerParams(
            dimension_semantics=("parallel",)),
    )(pk, t)

    table = paf_pred.reshape(B * C * 2, H * W)
    gxf, gyf = _sc_gather2(table, idx.reshape(B * C, _S * 8 * 128), 2)
    gx = gxf.reshape(B, C, _S, 8, 128)
    gy = gyf.reshape(B, C, _S, 8, 128)

    conn_p = pl.pallas_call(
        _paf_score_kernel,
        grid=(B,),
        in_specs=[
            pl.BlockSpec((1, C, 6, 8, 128), lambda b: (b, 0, 0, 0, 0)),
            pl.BlockSpec((1, C, _S, 8, 128), lambda b: (b, 0, 0, 0, 0)),
            pl.BlockSpec((1, C, _S, 8, 128), lambda b: (b, 0, 0, 0, 0)),
        ],
        out_specs=pl.BlockSpec((1, C, 8, 128), lambda b: (b, 0, 0, 0)),
        out_shape=jax.ShapeDtypeStruct((B, C, 8, 128), jnp.float32),
        compiler_params=pltpu.CompilerParams(
            dimension_semantics=("parallel",)),
    )(pk, gx, gy)

    conn = conn_p.reshape(B, C, 8 * 128)[:, :, :_P * _P].reshape(B, C, _P, _P)
    return px, py, topv, valid, conn
